# baseline (speedup 1.0000x reference)
import jax
import jax.numpy as jnp
from jax import lax
from jax.experimental import pallas as pl
from jax.experimental.pallas import tpu as pltpu
from jax.experimental.pallas import tpu_sc as plsc

NUM_EMB = 1000000
DIM = 64
NBLOCKS = NUM_EMB * DIM // 4096
NBLOCKS_PAD = 15632
NC = 2
NS = 16
NW = NC * NS
B = 16384 * 20
B_PER_W = B // NW
CHUNK = 128
N_CHUNKS = B_PER_W // CHUNK


def _sc_body(table_hbm, idx_hbm, absmax_hbm, out_hbm,
             idx_v, vidx_v, rows_v, scales_v, mshift_v, out_v, absmax_v, sem):
    wid = lax.axis_index("s") * NC + lax.axis_index("c")
    pltpu.sync_copy(absmax_hbm, absmax_v)
    pltpu.sync_copy(idx_hbm.at[wid], idx_v)

    def chunk_body(j, carry):
        for i in range(CHUNK // 16):
            iv = idx_v[j, pl.ds(i * 16, 16)]
            vidx_v[pl.ds(i * 16, 16)] = ((iv >> 6) << 4) | (iv & 15)
            mshift_v[pl.ds(i * 16, 16)] = 24 - 8 * ((iv >> 4) & 3)
            sc = plsc.load_gather(absmax_v, [iv >> 6]) * (1.0 / 127.0)
            scales_v[pl.ds(i * 16, 16)] = sc
        pltpu.async_copy(table_hbm.at[vidx_v], rows_v, sem).wait()

        def row_body(r, c):
            r16 = jnp.broadcast_to(r, (16,))
            s = plsc.load_gather(scales_v, [r16])
            lsh = plsc.load_gather(mshift_v, [r16])
            base = r * DIM
            for k in range(4):
                w = rows_v[r, pl.ds(k * 16, 16)]
                v = (w << lsh) >> 24
                out_v[pl.ds(base + k * 16, 16)] = v.astype(jnp.float32) * s
            return c

        lax.fori_loop(0, CHUNK, row_body, 0)
        pltpu.sync_copy(out_v, out_hbm.at[wid, j])
        return carry

    lax.fori_loop(0, N_CHUNKS, chunk_body, 0)


def _sc_call(table, idx, absmax_p):
    mesh = plsc.VectorSubcoreMesh(core_axis_name="c", subcore_axis_name="s",
                                  num_cores=NC, num_subcores=NS)
    return pl.kernel(
        _sc_body,
        out_type=jax.ShapeDtypeStruct((NW, N_CHUNKS, CHUNK * DIM), jnp.float32),
        mesh=mesh,
        scratch_types=[
            pltpu.VMEM((N_CHUNKS, CHUNK), jnp.int32),
            pltpu.VMEM((CHUNK,), jnp.int32),
            pltpu.VMEM((CHUNK, DIM), jnp.int32),
            pltpu.VMEM((CHUNK,), jnp.float32),
            pltpu.VMEM((CHUNK,), jnp.int32),
            pltpu.VMEM((CHUNK * DIM,), jnp.float32),
            pltpu.VMEM((NBLOCKS_PAD,), jnp.float32),
            pltpu.SemaphoreType.DMA,
        ],
        compiler_params=pltpu.CompilerParams(needs_layout_passes=False,
                                             use_tc_tiling_on_sc=False),
    )(table, idx, absmax_p)


@jax.jit
def kernel(q_weight, absmax, x):
    p0 = q_weight[:, 0:1024].astype(jnp.int32) & 0xFF
    p1 = q_weight[:, 1024:2048].astype(jnp.int32) & 0xFF
    p2 = q_weight[:, 2048:3072].astype(jnp.int32) & 0xFF
    p3 = q_weight[:, 3072:4096].astype(jnp.int32)
    packed = p0 | (p1 << 8) | (p2 << 16) | (p3 << 24)
    table = packed.reshape(NUM_EMB // 4, DIM)
    absmax_p = jnp.pad(absmax, (0, NBLOCKS_PAD - NBLOCKS))
    idx = x.reshape(NW, N_CHUNKS, CHUNK)
    out = _sc_call(table, idx, absmax_p)
    return out.reshape(x.shape[0], x.shape[1], DIM)


# speedup vs baseline: 1.2156x; 1.2156x over previous
import jax
import jax.numpy as jnp
from jax import lax
from jax.experimental import pallas as pl
from jax.experimental.pallas import tpu as pltpu
from jax.experimental.pallas import tpu_sc as plsc

NUM_EMB = 1000000
DIM = 64
NBLOCKS = NUM_EMB * DIM // 4096
NBLOCKS_PAD = 15632
NC = 2
NS = 16
NW = NC * NS
B = 16384 * 20
B_PER_W = B // NW
CHUNK = 128
N_CHUNKS = B_PER_W // CHUNK
SCALE = 1.0 / (127.0 * float(1 << 24))


def _sc_body(table_hbm, idx_hbm, absmax_hbm, out_hbm,
             idx_v, vidx_v, rows_v, scales_v, mshift_v, out_v, absmax_v,
             gsem, osem):
    wid = lax.axis_index("s") * NC + lax.axis_index("c")
    pltpu.sync_copy(absmax_hbm, absmax_v)
    pltpu.sync_copy(idx_hbm.at[wid], idx_v)

    def prepass(j, p):
        for i in range(CHUNK // 16):
            iv = idx_v[j, pl.ds(i * 16, 16)]
            vidx_v[p, pl.ds(i * 16, 16)] = ((iv >> 6) << 4) | (iv & 15)
            mshift_v[p, pl.ds(i * 16, 16)] = 24 - 8 * ((iv >> 4) & 3)
            sc = plsc.load_gather(absmax_v, [iv >> 6]) * SCALE
            scales_v[p, pl.ds(i * 16, 16)] = sc

    def gather(p):
        pltpu.async_copy(table_hbm.at[vidx_v.at[p]], rows_v.at[p], gsem)

    prepass(0, 0)
    gather(0)

    def chunk_body(j, carry):
        b = lax.rem(j, 2)
        nb = 1 - b

        @pl.when(j < N_CHUNKS - 1)
        def _():
            prepass(j + 1, nb)
            gather(nb)

        pltpu.make_async_copy(table_hbm.at[vidx_v.at[b]], rows_v.at[b],
                              gsem).wait()

        def row_body(r, c):
            r16 = jnp.broadcast_to(r, (16,))
            s = plsc.load_gather(scales_v.at[b], [r16])
            lsh = plsc.load_gather(mshift_v.at[b], [r16])
            base = r * DIM
            for k in range(4):
                w = rows_v[b, r, pl.ds(k * 16, 16)]
                out_v[b, pl.ds(base + k * 16, 16)] = \
                    (w << lsh).astype(jnp.float32) * s
            return c

        lax.fori_loop(0, CHUNK, row_body, 0, unroll=4)

        @pl.when(j >= 2)
        def _():
            pltpu.make_async_copy(out_v.at[b], out_hbm.at[wid, j - 2],
                                  osem).wait()

        pltpu.async_copy(out_v.at[b], out_hbm.at[wid, j], osem)
        return carry

    lax.fori_loop(0, N_CHUNKS, chunk_body, 0)
    pltpu.make_async_copy(out_v.at[0], out_hbm.at[wid, 0], osem).wait()
    pltpu.make_async_copy(out_v.at[1], out_hbm.at[wid, 0], osem).wait()


def _sc_call(table, idx, absmax_p):
    mesh = plsc.VectorSubcoreMesh(core_axis_name="c", subcore_axis_name="s",
                                  num_cores=NC, num_subcores=NS)
    return pl.kernel(
        _sc_body,
        out_type=jax.ShapeDtypeStruct((NW, N_CHUNKS, CHUNK * DIM), jnp.float32),
        mesh=mesh,
        scratch_types=[
            pltpu.VMEM((N_CHUNKS, CHUNK), jnp.int32),
            pltpu.VMEM((2, CHUNK), jnp.int32),
            pltpu.VMEM((2, CHUNK, DIM), jnp.int32),
            pltpu.VMEM((2, CHUNK), jnp.float32),
            pltpu.VMEM((2, CHUNK), jnp.int32),
            pltpu.VMEM((2, CHUNK * DIM), jnp.float32),
            pltpu.VMEM((NBLOCKS_PAD,), jnp.float32),
            pltpu.SemaphoreType.DMA,
            pltpu.SemaphoreType.DMA,
        ],
        compiler_params=pltpu.CompilerParams(needs_layout_passes=False,
                                             use_tc_tiling_on_sc=False),
    )(table, idx, absmax_p)


@jax.jit
def kernel(q_weight, absmax, x):
    p0 = q_weight[:, 0:1024].astype(jnp.int32) & 0xFF
    p1 = q_weight[:, 1024:2048].astype(jnp.int32) & 0xFF
    p2 = q_weight[:, 2048:3072].astype(jnp.int32) & 0xFF
    p3 = q_weight[:, 3072:4096].astype(jnp.int32)
    packed = p0 | (p1 << 8) | (p2 << 16) | (p3 << 24)
    table = packed.reshape(NUM_EMB // 4, DIM)
    absmax_p = jnp.pad(absmax, (0, NBLOCKS_PAD - NBLOCKS))
    idx = x.reshape(NW, N_CHUNKS, CHUNK)
    out = _sc_call(table, idx, absmax_p)
    return out.reshape(x.shape[0], x.shape[1], DIM)


# speedup vs baseline: 1.2179x; 1.0018x over previous
import jax
import jax.numpy as jnp
from jax import lax
from jax.experimental import pallas as pl
from jax.experimental.pallas import tpu as pltpu
from jax.experimental.pallas import tpu_sc as plsc

NUM_EMB = 1000000
DIM = 64
NBLOCKS = NUM_EMB * DIM // 4096
NBLOCKS_PAD = 15632
NC = 2
NS = 16
NW = NC * NS
B = 16384 * 20
B_PER_W = B // NW
CHUNK = 128
N_CHUNKS = B_PER_W // CHUNK
SCALE = 1.0 / (127.0 * float(1 << 24))


def _sc_body(table_hbm, idx_hbm, absmax_hbm, out_hbm,
             idx_v, vidx_v, rows_v, scales_v, mshift_v, out_v, absmax_v,
             gsem, osem):
    wid = lax.axis_index("s") * NC + lax.axis_index("c")
    pltpu.sync_copy(absmax_hbm, absmax_v)
    pltpu.sync_copy(idx_hbm.at[wid], idx_v)

    def prepass(j, p):
        for i in range(CHUNK // 16):
            iv = idx_v[j, pl.ds(i * 16, 16)]
            vidx_v[p, pl.ds(i * 16, 16)] = ((iv >> 6) << 4) | (iv & 15)
            mshift_v[p, pl.ds(i * 16, 16)] = 24 - 8 * ((iv >> 4) & 3)
            sc = plsc.load_gather(absmax_v, [iv >> 6]) * SCALE
            scales_v[p, pl.ds(i * 16, 16)] = sc

    def gather(p):
        pltpu.async_copy(table_hbm.at[vidx_v.at[p]], rows_v.at[p], gsem)

    prepass(0, 0)
    gather(0)

    def chunk_body(j, carry):
        b = lax.rem(j, 2)
        nb = 1 - b

        @pl.when(j < N_CHUNKS - 1)
        def _():
            prepass(j + 1, nb)
            gather(nb)

        pltpu.make_async_copy(table_hbm.at[vidx_v.at[b]], rows_v.at[b],
                              gsem).wait()

        @pl.when(j >= 2)
        def _():
            pltpu.make_async_copy(out_v.at[b], out_hbm.at[wid, j - 2],
                                  osem).wait()

        def row_body(r, c):
            r16 = jnp.broadcast_to(r, (16,))
            s = plsc.load_gather(scales_v.at[b], [r16])
            lsh = plsc.load_gather(mshift_v.at[b], [r16])
            base = r * DIM
            for k in range(4):
                w = rows_v[b, r, pl.ds(k * 16, 16)]
                out_v[b, pl.ds(base + k * 16, 16)] = \
                    (w << lsh).astype(jnp.float32) * s
            return c

        lax.fori_loop(0, CHUNK, row_body, 0, unroll=4)
        pltpu.async_copy(out_v.at[b], out_hbm.at[wid, j], osem)
        return carry

    lax.fori_loop(0, N_CHUNKS, chunk_body, 0)
    pltpu.make_async_copy(out_v.at[0], out_hbm.at[wid, 0], osem).wait()
    pltpu.make_async_copy(out_v.at[1], out_hbm.at[wid, 0], osem).wait()


def _sc_call(table, idx, absmax_p):
    mesh = plsc.VectorSubcoreMesh(core_axis_name="c", subcore_axis_name="s",
                                  num_cores=NC, num_subcores=NS)
    return pl.kernel(
        _sc_body,
        out_type=jax.ShapeDtypeStruct((NW, N_CHUNKS, CHUNK * DIM), jnp.float32),
        mesh=mesh,
        scratch_types=[
            pltpu.VMEM((N_CHUNKS, CHUNK), jnp.int32),
            pltpu.VMEM((2, CHUNK), jnp.int32),
            pltpu.VMEM((2, CHUNK, DIM), jnp.int32),
            pltpu.VMEM((2, CHUNK), jnp.float32),
            pltpu.VMEM((2, CHUNK), jnp.int32),
            pltpu.VMEM((2, CHUNK * DIM), jnp.float32),
            pltpu.VMEM((NBLOCKS_PAD,), jnp.float32),
            pltpu.SemaphoreType.DMA,
            pltpu.SemaphoreType.DMA,
        ],
        compiler_params=pltpu.CompilerParams(needs_layout_passes=False,
                                             use_tc_tiling_on_sc=False),
    )(table, idx, absmax_p)


@jax.jit
def kernel(q_weight, absmax, x):
    p0 = q_weight[:, 0:1024].astype(jnp.int32) & 0xFF
    p1 = q_weight[:, 1024:2048].astype(jnp.int32) & 0xFF
    p2 = q_weight[:, 2048:3072].astype(jnp.int32) & 0xFF
    p3 = q_weight[:, 3072:4096].astype(jnp.int32)
    packed = p0 | (p1 << 8) | (p2 << 16) | (p3 << 24)
    table = packed.reshape(NUM_EMB // 4, DIM)
    absmax_p = jnp.pad(absmax, (0, NBLOCKS_PAD - NBLOCKS))
    idx = x.reshape(NW, N_CHUNKS, CHUNK)
    out = _sc_call(table, idx, absmax_p)
    return out.reshape(x.shape[0], x.shape[1], DIM)
